# baseline (speedup 1.0000x reference)
import functools

import jax
import jax.numpy as jnp
import numpy as np
from jax.experimental import pallas as pl
from jax.experimental.pallas import tpu as pltpu


def _mp_body(h_ref, a_ref, e_ref, wcat_ref, we_ref, r_ref, s_ref, b_ref,
             o_ref, *, d_node):
    h = h_ref[0]
    a = a_ref[0]
    e = e_ref[0]

    hw = jnp.dot(h, wcat_ref[...], preferred_element_type=jnp.float32)
    hs = hw[:, :d_node]
    hn = hw[:, d_node:]
    deg = jnp.sum(a, axis=1, keepdims=True)
    arep = jnp.dot(a, r_ref[...], preferred_element_type=jnp.float32)
    p = arep * e
    m = jnp.dot(p, s_ref[...], preferred_element_type=jnp.float32)
    me = jnp.dot(m, we_ref[...], preferred_element_type=jnp.float32)
    o_ref[0] = hs + deg * hn + me + b_ref[...]


def kernel(H, A, E, N, W_self, W_nei, bias):
    B, Nn, D = H.shape
    De = E.shape[-1]
    E2 = E.reshape(B, Nn, Nn * De)
    W_cat = jnp.concatenate([W_self, W_nei[:D]], axis=1)
    W_e = W_nei[D:]
    R = jnp.asarray(np.kron(np.eye(Nn, dtype=np.float32),
                            np.ones((1, De), np.float32)))
    S = jnp.asarray(np.kron(np.ones((Nn, 1), np.float32),
                            np.eye(De, dtype=np.float32)))
    bias2 = bias[None, :]

    grid = (B,)
    out = pl.pallas_call(
        functools.partial(_mp_body, d_node=D),
        grid=grid,
        in_specs=[
            pl.BlockSpec((1, Nn, D), lambda a: (a, 0, 0)),
            pl.BlockSpec((1, Nn, Nn), lambda a: (a, 0, 0)),
            pl.BlockSpec((1, Nn, Nn * De), lambda a: (a, 0, 0)),
            pl.BlockSpec((D, 2 * D), lambda a: (0, 0)),
            pl.BlockSpec((De, D), lambda a: (0, 0)),
            pl.BlockSpec((Nn, Nn * De), lambda a: (0, 0)),
            pl.BlockSpec((Nn * De, De), lambda a: (0, 0)),
            pl.BlockSpec((1, D), lambda a: (0, 0)),
        ],
        out_specs=pl.BlockSpec((1, Nn, D), lambda a: (a, 0, 0)),
        out_shape=jax.ShapeDtypeStruct((B, Nn, D), jnp.float32),
        compiler_params=pltpu.CompilerParams(
            dimension_semantics=("arbitrary",),
        ),
    )(H, A, E2, W_cat, W_e, R, S, bias2)
    return out
